# baseline (speedup 1.0000x reference)
import functools

import jax
import jax.numpy as jnp
from jax import lax
from jax.experimental import pallas as pl
from jax.experimental.pallas import tpu as pltpu
from jax.experimental.pallas import tpu_sc as plsc

BATCH = 4096
NFIELD = 282
NPOS = 283
DIM = 64
NTAB = 790
NC = 2
NS = 16
NW = NC * NS
BLK = BATCH // NW
NG = BLK // 16
BUFW = (DIM // 8) * 1024


def _body(x_hbm, starts_hbm, table_hbm, out_hbm,
          x_blk, table_v, se_v, buf, wsem_a, wsem_b):
    wid = lax.axis_index("s") * NC + lax.axis_index("c")
    bbase = wid * BLK

    pltpu.sync_copy(x_hbm.at[pl.ds(bbase * NPOS, BLK * NPOS)], x_blk)
    pltpu.sync_copy(table_hbm, table_v)
    pltpu.sync_copy(starts_hbm, se_v.at[pl.ds(0, NPOS)])

    iota283 = lax.iota(jnp.int32, 16) * NPOS

    def q_body(q, carry):
        p = q >> 3
        g = q & 7
        par = p & 1
        boff = par * BUFW

        @pl.when((g == 0) & (p >= 2) & (par == 0))
        def _():
            for d8 in range(DIM // 8):
                pltpu.make_async_copy(buf.at[pl.ds(0, 1024)],
                                      out_hbm.at[pl.ds(0, 1024)],
                                      wsem_a).wait()

        @pl.when((g == 0) & (p >= 2) & (par == 1))
        def _():
            for d8 in range(DIM // 8):
                pltpu.make_async_copy(buf.at[pl.ds(0, 1024)],
                                      out_hbm.at[pl.ds(0, 1024)],
                                      wsem_b).wait()

        soff16 = plsc.load_gather(se_v, [jnp.full((16,), p, jnp.int32)])
        xat16 = iota283 + (g * 16 * NPOS + p)
        x16 = plsc.load_gather(x_blk, [xat16])
        base16 = (x16 + soff16) * DIM

        col = boff + g * 16
        for d0 in range(0, DIM, 4):
            vals = [plsc.load_gather(table_v, [base16 + (d0 + k)])
                    for k in range(4)]
            for k in range(4):
                d = d0 + k
                buf[pl.ds(col + (d // 8) * 1024 + (d % 8) * 128, 16)] = \
                    vals[k]

        out0 = p * (DIM * BATCH) + wid * 1024

        @pl.when((g == 7) & (par == 0))
        def _():
            for d8 in range(DIM // 8):
                pltpu.async_copy(
                    buf.at[pl.ds(boff + d8 * 1024, 1024)],
                    out_hbm.at[pl.ds(out0 + d8 * (NW * 1024), 1024)],
                    wsem_a)

        @pl.when((g == 7) & (par == 1))
        def _():
            for d8 in range(DIM // 8):
                pltpu.async_copy(
                    buf.at[pl.ds(boff + d8 * 1024, 1024)],
                    out_hbm.at[pl.ds(out0 + d8 * (NW * 1024), 1024)],
                    wsem_b)

        return carry

    lax.fori_loop(0, NPOS * NG, q_body, 0)
    for d8 in range(DIM // 8):
        pltpu.make_async_copy(buf.at[pl.ds(0, 1024)],
                              out_hbm.at[pl.ds(0, 1024)], wsem_a).wait()
        pltpu.make_async_copy(buf.at[pl.ds(0, 1024)],
                              out_hbm.at[pl.ds(0, 1024)], wsem_b).wait()


@functools.partial(jax.jit, static_argnames=())
def _emb_lookup(x, starts, starter, table):
    mesh = plsc.VectorSubcoreMesh(core_axis_name="c", subcore_axis_name="s")
    run = functools.partial(
        pl.kernel,
        mesh=mesh,
        out_type=jax.ShapeDtypeStruct((NPOS * DIM * BATCH,), jnp.float32),
        compiler_params=pltpu.CompilerParams(use_tc_tiling_on_sc=False,
                                             needs_layout_passes=False),
        scratch_types=[
            pltpu.VMEM((BLK * NPOS,), jnp.int32),
            pltpu.VMEM((NTAB * DIM,), jnp.float32),
            pltpu.VMEM((288,), jnp.int32),
            pltpu.VMEM((2 * BUFW,), jnp.float32),
            pltpu.SemaphoreType.DMA,
            pltpu.SemaphoreType.DMA,
        ],
    )(_body)
    xz = jnp.concatenate(
        [jnp.zeros((BATCH, 1), jnp.int32), x], axis=1).reshape(-1)
    starts_ext = jnp.concatenate([starter, starts])
    kout = run(xz, starts_ext, table.reshape(-1))
    k5 = kout.reshape(NPOS, DIM // 8, NW, 8, 128)
    return k5.transpose(2, 4, 0, 1, 3).reshape(BATCH, NPOS, DIM)


def kernel(x, field_start_idx_array, starter_idx_array, table):
    return _emb_lookup(x, field_start_idx_array, starter_idx_array, table)
